# Initial kernel scaffold; baseline (speedup 1.0000x reference)
import jax
import jax.numpy as jnp
from jax.experimental import pallas as pl


def kernel(hidden_states, router_w, router_b, w_gate_up, b_gate_up, w_down, b_down):
    raise NotImplementedError("write your pallas kernel here")


# speedup vs baseline: 10.7026x; 10.7026x over previous
import functools

import jax
import jax.numpy as jnp
from jax.experimental import pallas as pl
from jax.experimental.pallas import tpu as pltpu

E = 64
TOP_K = 2
D_MODEL = 1024
D_FF = 512
T = 256
ALPHA = 1.702
LIMIT = 7.0

NA = T * TOP_K
TILE = 8
MAX_ROWS = 1024

_HIGH = jax.lax.Precision.HIGHEST


def _routing_kernel(hid_ref, rw_ref, rb_ref, g_ref, gw_ref, offs_ref):
    hid = hid_ref[...]
    logits = jax.lax.dot_general(
        hid.astype(jnp.bfloat16), rw_ref[...].astype(jnp.bfloat16),
        (((1,), (1,)), ((), ())),
        preferred_element_type=jnp.float32,
    ) + rb_ref[...]

    idx = jax.lax.broadcasted_iota(jnp.int32, (T, E), 1)
    m1 = jnp.max(logits, axis=1, keepdims=True)
    a1 = jnp.min(jnp.where(logits == m1, idx, E), axis=1, keepdims=True)
    neg = jnp.where(idx == a1, -1e30, logits)
    m2 = jnp.max(neg, axis=1, keepdims=True)
    a2 = jnp.min(jnp.where(neg == m2, idx, E), axis=1, keepdims=True)
    w1 = 1.0 / (1.0 + jnp.exp(m2 - m1))
    w2 = 1.0 - w1

    ef = jnp.concatenate([a1, a2], axis=0)
    wf = jnp.concatenate([w1, w2], axis=0)

    eiota = jax.lax.broadcasted_iota(jnp.int32, (NA, E), 1)
    M = (ef == eiota).astype(jnp.float32)
    counts = jnp.sum(M, axis=0, keepdims=True)
    ntiles = jnp.floor((counts + 7.0) * 0.125)

    r64 = jax.lax.broadcasted_iota(jnp.int32, (E, E), 0)
    c64 = jax.lax.broadcasted_iota(jnp.int32, (E, E), 1)
    U64 = (r64 <= c64).astype(jnp.float32)
    cumt = jax.lax.dot_general(ntiles, U64, (((1,), (0,)), ((), ())),
                               preferred_element_type=jnp.float32,
                               precision=_HIGH)
    offs8 = 8.0 * (cumt - ntiles)

    rA = jax.lax.broadcasted_iota(jnp.int32, (NA, NA), 0)
    cA = jax.lax.broadcasted_iota(jnp.int32, (NA, NA), 1)
    Lstrict = (cA < rA).astype(jnp.float32)
    C_ex = jax.lax.dot_general(Lstrict, M, (((1,), (0,)), ((), ())),
                               preferred_element_type=jnp.float32,
                               precision=_HIGH)
    rank = jnp.sum(M * C_ex, axis=1, keepdims=True)
    base = jnp.sum(M * offs8, axis=1, keepdims=True)
    pos = rank + base

    prows = jax.lax.broadcasted_iota(jnp.int32, (NA, MAX_ROWS), 1)
    P1t = (pos.astype(jnp.int32) == prows).astype(jnp.float32)
    tok = jnp.concatenate(
        [jax.lax.broadcasted_iota(jnp.int32, (T, 1), 0)] * 2, axis=0)
    tcols = jax.lax.broadcasted_iota(jnp.int32, (NA, T), 1)
    Tm = (tok == tcols).astype(jnp.float32)

    g_ref[...] = jax.lax.dot_general(P1t, Tm, (((0,), (0,)), ((), ())),
                                     preferred_element_type=jnp.float32,
                                     precision=_HIGH)
    gw_ref[...] = jax.lax.dot_general(P1t, Tm * wf, (((0,), (0,)), ((), ())),
                                      preferred_element_type=jnp.float32,
                                      precision=_HIGH)
    offs_ref[...] = jnp.broadcast_to(cumt.astype(jnp.int32), (8, E))


def _expert_kernel(offs_ref, hid_ref, wg_ref, bg_ref, wd_ref, bd_ref,
                   g_ref, gw_ref, out_ref, y_ref):
    e = pl.program_id(0)

    @pl.when(e == 0)
    def _init():
        y_ref[...] = jnp.zeros_like(y_ref)

    end = offs_ref[e]
    start = jnp.where(e == 0, 0, offs_ref[jnp.maximum(e - 1, 0)])
    ntile = end - start

    hid = hid_ref[...]
    wg = wg_ref[0]
    bg = bg_ref[0]
    wd = wd_ref[0]
    bd = bd_ref[0]

    def body(i, _):
        r0 = (start + i) * TILE
        gt = g_ref[pl.ds(r0, TILE), :]
        x = jnp.dot(gt, hid, preferred_element_type=jnp.float32)
        gu = jnp.dot(x, wg, preferred_element_type=jnp.float32) + bg
        gu2 = gu.reshape(TILE, D_FF, 2)
        gate = jnp.minimum(gu2[:, :, 0], LIMIT)
        up = jnp.clip(gu2[:, :, 1], -LIMIT, LIMIT)
        glu = gate * jax.nn.sigmoid(ALPHA * gate)
        act = (up + 1.0) * glu
        y = jnp.dot(act, wd, preferred_element_type=jnp.float32) + bd
        gwt = gw_ref[pl.ds(r0, TILE), :]
        w8 = jnp.sum(gwt, axis=1, keepdims=True)
        y_ref[pl.ds(r0, TILE), :] = y * w8
        return 0

    jax.lax.fori_loop(0, ntile, body, 0)

    @pl.when(e == E - 1)
    def _combine():
        out_ref[...] = jax.lax.dot_general(
            g_ref[...], y_ref[...], (((0,), (0,)), ((), ())),
            preferred_element_type=jnp.float32)


@functools.partial(jax.jit, static_argnames=())
def kernel(hidden_states, router_w, router_b, w_gate_up, b_gate_up,
           w_down, b_down):
    g, gw, offs = pl.pallas_call(
        _routing_kernel,
        out_shape=[
            jax.ShapeDtypeStruct((MAX_ROWS, T), jnp.float32),
            jax.ShapeDtypeStruct((MAX_ROWS, T), jnp.float32),
            jax.ShapeDtypeStruct((8, E), jnp.int32),
        ],
    )(hidden_states, router_w, router_b.reshape(1, E))

    grid_spec = pltpu.PrefetchScalarGridSpec(
        num_scalar_prefetch=1,
        grid=(E,),
        in_specs=[
            pl.BlockSpec((T, D_MODEL), lambda e, s: (0, 0)),
            pl.BlockSpec((1, D_MODEL, 2 * D_FF), lambda e, s: (e, 0, 0)),
            pl.BlockSpec((1, 1, 2 * D_FF), lambda e, s: (e, 0, 0)),
            pl.BlockSpec((1, D_FF, D_MODEL), lambda e, s: (e, 0, 0)),
            pl.BlockSpec((1, 1, D_MODEL), lambda e, s: (e, 0, 0)),
            pl.BlockSpec((MAX_ROWS, T), lambda e, s: (0, 0)),
            pl.BlockSpec((MAX_ROWS, T), lambda e, s: (0, 0)),
        ],
        out_specs=pl.BlockSpec((T, D_MODEL), lambda e, s: (0, 0)),
        scratch_shapes=[pltpu.VMEM((MAX_ROWS, D_MODEL), jnp.float32)],
    )
    out = pl.pallas_call(
        _expert_kernel,
        grid_spec=grid_spec,
        out_shape=jax.ShapeDtypeStruct((T, D_MODEL), jnp.float32),
    )(offs[0], hidden_states, w_gate_up,
      b_gate_up.reshape(E, 1, 2 * D_FF), w_down,
      b_down.reshape(E, 1, D_MODEL), g, gw)
    return out
